# baseline (speedup 1.0000x reference)
import jax
import jax.numpy as jnp
from jax import lax
from jax.experimental import pallas as pl

MEL = 4096
TM = 512


def _layer_norm(h, g, b):
    mu = jnp.mean(h, axis=1, keepdims=True)
    var = jnp.mean((h - mu) ** 2, axis=1, keepdims=True)
    return (h - mu) / jnp.sqrt(var + 1e-5) * g + b


def _body(x_ref, t_ref, mml_ref,
          w1p, w1c, w1n, b1, g1, be1,
          w2p, w2c, w2n, b2, g2, be2,
          lw, lb,
          out_ref, al_ref, dp_ref):
    L = t_ref.shape[2]
    D = x_ref.shape[2]
    mt = pl.program_id(1)

    dur = t_ref[0].astype(jnp.float32)
    tri = (lax.broadcasted_iota(jnp.int32, (L, L), 0)
           <= lax.broadcasted_iota(jnp.int32, (L, L), 1)).astype(jnp.float32)
    csum = jnp.dot(dur, tri, preferred_element_type=jnp.float32)

    mvec = (mt * TM + lax.broadcasted_iota(jnp.int32, (TM, 1), 0)
            ).astype(jnp.float32)
    a = ((csum > mvec) & (mvec >= csum - dur)
         & (mvec < mml_ref[0, 0])).astype(jnp.float32)
    al_ref[0] = a
    out_ref[0] = jnp.dot(a, x_ref[0], preferred_element_type=jnp.float32)

    @pl.when(mt == 0)
    def _dp():
        xb = x_ref[0]
        zr = jnp.zeros((1, D), jnp.float32)
        xp = jnp.concatenate([zr, xb[:-1]], axis=0)
        xn = jnp.concatenate([xb[1:], zr], axis=0)
        h = (jnp.dot(xp, w1p[...], preferred_element_type=jnp.float32)
             + jnp.dot(xb, w1c[...], preferred_element_type=jnp.float32)
             + jnp.dot(xn, w1n[...], preferred_element_type=jnp.float32)
             + b1[...])
        h = jax.nn.relu(_layer_norm(h, g1[...], be1[...]))
        hp = jnp.concatenate([zr, h[:-1]], axis=0)
        hn = jnp.concatenate([h[1:], zr], axis=0)
        h2 = (jnp.dot(hp, w2p[...], preferred_element_type=jnp.float32)
              + jnp.dot(h, w2c[...], preferred_element_type=jnp.float32)
              + jnp.dot(hn, w2n[...], preferred_element_type=jnp.float32)
              + b2[...])
        h2 = jax.nn.relu(_layer_norm(h2, g2[...], be2[...]))
        dp = jnp.exp(jnp.sum(h2 * lw[...], axis=1) + lb[0, 0])
        dp_ref[0] = dp.reshape(1, L)


def kernel(x, target, mel_max_length,
           conv1_w, conv1_b, ln1_g, ln1_b,
           conv2_w, conv2_b, ln2_g, ln2_b,
           lin_w, lin_b):
    B, L, D = x.shape
    F = conv1_w.shape[0]
    t3 = target.reshape(B, 1, L)
    mml = jnp.asarray(mel_max_length, jnp.float32).reshape(1, 1)
    w1p = conv1_w[:, :, 0].T
    w1c = conv1_w[:, :, 1].T
    w1n = conv1_w[:, :, 2].T
    w2p = conv2_w[:, :, 0].T
    w2c = conv2_w[:, :, 1].T
    w2n = conv2_w[:, :, 2].T
    b1 = conv1_b.reshape(1, F)
    g1 = ln1_g.reshape(1, F)
    be1 = ln1_b.reshape(1, F)
    b2 = conv2_b.reshape(1, F)
    g2 = ln2_g.reshape(1, F)
    be2 = ln2_b.reshape(1, F)
    lw = lin_w.reshape(1, F)
    lb = lin_b.reshape(1, 1)

    const = lambda *_: (0, 0)
    out, align, dp3 = pl.pallas_call(
        _body,
        grid=(B, MEL // TM),
        in_specs=[
            pl.BlockSpec((1, L, D), lambda b, mt: (b, 0, 0)),
            pl.BlockSpec((1, 1, L), lambda b, mt: (b, 0, 0)),
            pl.BlockSpec((1, 1), const),
            pl.BlockSpec((D, F), const), pl.BlockSpec((D, F), const),
            pl.BlockSpec((D, F), const), pl.BlockSpec((1, F), const),
            pl.BlockSpec((1, F), const), pl.BlockSpec((1, F), const),
            pl.BlockSpec((F, F), const), pl.BlockSpec((F, F), const),
            pl.BlockSpec((F, F), const), pl.BlockSpec((1, F), const),
            pl.BlockSpec((1, F), const), pl.BlockSpec((1, F), const),
            pl.BlockSpec((1, F), const), pl.BlockSpec((1, 1), const),
        ],
        out_specs=[
            pl.BlockSpec((1, TM, D), lambda b, mt: (b, mt, 0)),
            pl.BlockSpec((1, TM, L), lambda b, mt: (b, mt, 0)),
            pl.BlockSpec((1, 1, L), lambda b, mt: (b, 0, 0)),
        ],
        out_shape=[
            jax.ShapeDtypeStruct((B, MEL, D), jnp.float32),
            jax.ShapeDtypeStruct((B, MEL, L), jnp.float32),
            jax.ShapeDtypeStruct((B, 1, L), jnp.float32),
        ],
    )(x, t3, mml, w1p, w1c, w1n, b1, g1, be1,
      w2p, w2c, w2n, b2, g2, be2, lw, lb)
    return (out, align, dp3.reshape(B, L))
